# baseline (speedup 1.0000x reference)
import functools

import jax
import jax.numpy as jnp
from jax import lax
from jax.experimental import pallas as pl
from jax.experimental.pallas import tpu as pltpu
from jax.experimental.pallas import tpu_sc as plsc

B = 16384
D = 64
L = 16

_info = plsc.get_sparse_core_info()
NC = _info.num_cores
NS = _info.num_subcores
NW = NC * NS
PW = B // NW
CH = 128
NCH = PW // CH
GROUPS = PW // L


def _trans_e_body(h_hbm, r_hbm, t_hbm, ent_hbm, rel_hbm, out_hbm,
                  hi, ri, ti, hrows, rrows, trows, outv, sem):
    wid = lax.axis_index("s") * NC + lax.axis_index("c")

    pltpu.sync_copy(h_hbm.at[wid], hi)
    pltpu.sync_copy(r_hbm.at[wid], ri)
    pltpu.sync_copy(t_hbm.at[wid], ti)

    copies = []
    for j in range(NCH):
        copies.append(pltpu.async_copy(
            ent_hbm.at[hi.at[j]], hrows.at[pl.ds(j * CH, CH)], sem))
        copies.append(pltpu.async_copy(
            rel_hbm.at[ri.at[j]], rrows.at[pl.ds(j * CH, CH)], sem))
        copies.append(pltpu.async_copy(
            ent_hbm.at[ti.at[j]], trows.at[pl.ds(j * CH, CH)], sem))
    for c in copies:
        c.wait()

    lane = lax.iota(jnp.int32, L)

    def group_body(g, _):
        row = lane + g * L
        acc = jnp.zeros((L,), jnp.float32)
        for c in range(D):
            col = jnp.full((L,), c, jnp.int32)
            hv = plsc.load_gather(hrows, [row, col])
            rv = plsc.load_gather(rrows, [row, col])
            tv = plsc.load_gather(trows, [row, col])
            acc = acc + jnp.abs(hv + rv - tv)
        outv[pl.ds(g * L, L)] = acc
        return 0

    lax.fori_loop(0, GROUPS, group_body, 0)

    pltpu.sync_copy(outv, out_hbm.at[pl.ds(wid * PW, PW)])


@jax.jit
def _trans_e(h, r, t, entity_emb, relation_emb):
    mesh = plsc.VectorSubcoreMesh(core_axis_name="c", subcore_axis_name="s")
    run = functools.partial(
        pl.kernel,
        mesh=mesh,
        compiler_params=pltpu.CompilerParams(
            needs_layout_passes=False, use_tc_tiling_on_sc=False),
        out_type=jax.ShapeDtypeStruct((B,), jnp.float32),
        scratch_types=[
            pltpu.VMEM((NCH, CH), jnp.int32),
            pltpu.VMEM((NCH, CH), jnp.int32),
            pltpu.VMEM((NCH, CH), jnp.int32),
            pltpu.VMEM((PW, D), jnp.float32),
            pltpu.VMEM((PW, D), jnp.float32),
            pltpu.VMEM((PW, D), jnp.float32),
            pltpu.VMEM((PW,), jnp.float32),
            pltpu.SemaphoreType.DMA,
        ],
    )(_trans_e_body)
    return run(h, r, t, entity_emb, relation_emb)


def kernel(h, r, t, entity_emb, relation_emb):
    h3 = h.astype(jnp.int32).reshape(NW, NCH, CH)
    r3 = r.astype(jnp.int32).reshape(NW, NCH, CH)
    t3 = t.astype(jnp.int32).reshape(NW, NCH, CH)
    return _trans_e(h3, r3, t3, entity_emb, relation_emb)


# speedup vs baseline: 1.0055x; 1.0055x over previous
import functools

import jax
import jax.numpy as jnp
from jax import lax
from jax.experimental import pallas as pl
from jax.experimental.pallas import tpu as pltpu
from jax.experimental.pallas import tpu_sc as plsc

B = 16384
D = 64
L = 16
W = 128

_info = plsc.get_sparse_core_info()
NC = _info.num_cores
NS = _info.num_subcores
NW = NC * NS
PW = B // NW
CH = 128
HALF = PW // 2
NCH = HALF // CH
HGRP = HALF // L


def _trans_e_body(hp_hbm, ho_hbm, rp_hbm, ro_hbm, tp_hbm, to_hbm,
                  ent_hbm, rel_hbm, out_hbm,
                  hi, ri, ti, hoff, roff, toff,
                  hrows, rrows, trows, outv, sem):
    wid = lax.axis_index("s") * NC + lax.axis_index("c")

    pltpu.sync_copy(hp_hbm.at[wid], hi)
    pltpu.sync_copy(ho_hbm.at[wid], hoff)
    pltpu.sync_copy(rp_hbm.at[wid], ri)
    pltpu.sync_copy(ro_hbm.at[wid], roff)
    pltpu.sync_copy(tp_hbm.at[wid], ti)
    pltpu.sync_copy(to_hbm.at[wid], toff)

    lane = lax.iota(jnp.int32, L)

    for half in range(2):
        copies = []
        for j in range(NCH):
            jc = half * NCH + j
            copies.append(pltpu.async_copy(
                ent_hbm.at[hi.at[jc]], hrows.at[pl.ds(j * CH, CH)], sem))
            copies.append(pltpu.async_copy(
                rel_hbm.at[ri.at[jc]], rrows.at[pl.ds(j * CH, CH)], sem))
            copies.append(pltpu.async_copy(
                ent_hbm.at[ti.at[jc]], trows.at[pl.ds(j * CH, CH)], sem))
        for c in copies:
            c.wait()

        def group_body(g, _, half=half):
            row = lane + g * L
            hov = hoff[pl.ds(half * HALF + g * L, L)]
            rov = roff[pl.ds(half * HALF + g * L, L)]
            tov = toff[pl.ds(half * HALF + g * L, L)]
            acc = jnp.zeros((L,), jnp.float32)
            for c in range(D):
                col = jnp.full((L,), c, jnp.int32)
                hv = plsc.load_gather(hrows, [row, col + hov])
                rv = plsc.load_gather(rrows, [row, col + rov])
                tv = plsc.load_gather(trows, [row, col + tov])
                acc = acc + jnp.abs(hv + rv - tv)
            outv[pl.ds(half * HALF + g * L, L)] = acc
            return 0

        lax.fori_loop(0, HGRP, group_body, 0)

    pltpu.sync_copy(outv, out_hbm.at[pl.ds(wid * PW, PW)])


@jax.jit
def _trans_e(hp, ho, rp, ro, tp, to, ent2, rel2):
    mesh = plsc.VectorSubcoreMesh(core_axis_name="c", subcore_axis_name="s")
    run = functools.partial(
        pl.kernel,
        mesh=mesh,
        compiler_params=pltpu.CompilerParams(needs_layout_passes=False),
        out_type=jax.ShapeDtypeStruct((B,), jnp.float32),
        scratch_types=[
            pltpu.VMEM((2 * NCH, CH), jnp.int32),
            pltpu.VMEM((2 * NCH, CH), jnp.int32),
            pltpu.VMEM((2 * NCH, CH), jnp.int32),
            pltpu.VMEM((PW,), jnp.int32),
            pltpu.VMEM((PW,), jnp.int32),
            pltpu.VMEM((PW,), jnp.int32),
            pltpu.VMEM((HALF, W), jnp.float32),
            pltpu.VMEM((HALF, W), jnp.float32),
            pltpu.VMEM((HALF, W), jnp.float32),
            pltpu.VMEM((PW,), jnp.float32),
            pltpu.SemaphoreType.DMA,
        ],
    )(_trans_e_body)
    return run(hp, ho, rp, ro, tp, to, ent2, rel2)


def kernel(h, r, t, entity_emb, relation_emb):
    h = h.astype(jnp.int32)
    r = r.astype(jnp.int32)
    t = t.astype(jnp.int32)
    hp = (h >> 1).reshape(NW, 2 * NCH, CH)
    ho = ((h & 1) * D).reshape(NW, PW)
    rp = (r >> 1).reshape(NW, 2 * NCH, CH)
    ro = ((r & 1) * D).reshape(NW, PW)
    tp = (t >> 1).reshape(NW, 2 * NCH, CH)
    to = ((t & 1) * D).reshape(NW, PW)
    ent2 = entity_emb.reshape(-1, W)
    rel2 = relation_emb.reshape(-1, W)
    return _trans_e(hp, ho, rp, ro, tp, to, ent2, rel2)


# speedup vs baseline: 1.4998x; 1.4915x over previous
import functools

import jax
import jax.numpy as jnp
from jax import lax
from jax.experimental import pallas as pl
from jax.experimental.pallas import tpu as pltpu
from jax.experimental.pallas import tpu_sc as plsc

B = 16384
D = 64
NR = 1000
L = 16

_info = plsc.get_sparse_core_info()
NC = _info.num_cores
NS = _info.num_subcores
NW = NC * NS
PW = B // NW
CPR = 128
NCK = PW // CPR
CGRP = CPR // L


def _trans_e_body(h_hbm, r_hbm, t_hbm, ent_hbm, rel_hbm, out_hbm,
                  him, rim, tim, hb, tb, rloc, outv, sem):
    wid = lax.axis_index("s") * NC + lax.axis_index("c")

    pltpu.sync_copy(h_hbm.at[wid], him)
    pltpu.sync_copy(r_hbm.at[wid], rim)
    pltpu.sync_copy(t_hbm.at[wid], tim)
    def rel_body(gg, _):
        pltpu.sync_copy(rel_hbm.at[pl.ds(gg * 8, 8)],
                        rloc.at[pl.ds(gg * 8, 8)])
        return 0

    lax.fori_loop(0, (NR // 2) // 8, rel_body, 0)
    pltpu.sync_copy(rel_hbm.at[pl.ds(496, 4)], rloc.at[pl.ds(496, 4)])

    lane = lax.iota(jnp.int32, L)

    def chunk_body(k, _):
        base = k * CPR

        def fire_body(q, _):
            hv16 = him[pl.ds(base + q * L, L)]
            tv16 = tim[pl.ds(base + q * L, L)]
            for jj in range(L):
                i = q * L + jj
                pltpu.async_copy(ent_hbm.at[hv16[jj]], hb.at[i], sem)
                pltpu.async_copy(ent_hbm.at[tv16[jj]], tb.at[i], sem)
            return 0

        lax.fori_loop(0, CPR // L, fire_body, 0)

        def drain_body(i, _):
            pltpu.make_async_copy(ent_hbm.at[0], hb.at[0], sem).wait()
            pltpu.make_async_copy(ent_hbm.at[0], tb.at[0], sem).wait()
            return 0

        lax.fori_loop(0, CPR, drain_body, 0)

        def group_body(g, _):
            row = lane + g * L
            rv16 = rim[pl.ds(base + g * L, L)]
            rp16 = jax.lax.shift_right_logical(rv16, 1)
            ro16 = (rv16 & 1) * D
            acc = jnp.zeros((L,), jnp.float32)
            for c in range(D):
                col = jnp.full((L,), c, jnp.int32)
                hv = plsc.load_gather(hb, [row, col])
                tv = plsc.load_gather(tb, [row, col])
                rv = plsc.load_gather(rloc, [rp16, col + ro16])
                acc = acc + jnp.abs(hv + rv - tv)
            outv[pl.ds(base + g * L, L)] = acc
            return 0

        lax.fori_loop(0, CGRP, group_body, 0)
        return 0

    lax.fori_loop(0, NCK, chunk_body, 0)

    pltpu.sync_copy(outv, out_hbm.at[pl.ds(wid * PW, PW)])


@jax.jit
def _trans_e(h, r, t, entity_emb, rel2):
    mesh = plsc.VectorSubcoreMesh(core_axis_name="c", subcore_axis_name="s")
    run = functools.partial(
        pl.kernel,
        mesh=mesh,
        compiler_params=pltpu.CompilerParams(needs_layout_passes=False),
        out_type=jax.ShapeDtypeStruct((B,), jnp.float32),
        scratch_types=[
            pltpu.VMEM((PW,), jnp.int32),
            pltpu.VMEM((PW,), jnp.int32),
            pltpu.VMEM((PW,), jnp.int32),
            pltpu.VMEM((CPR, D), jnp.float32),
            pltpu.VMEM((CPR, D), jnp.float32),
            pltpu.VMEM((NR // 2, 2 * D), jnp.float32),
            pltpu.VMEM((PW,), jnp.float32),
            pltpu.SemaphoreType.DMA,
        ],
    )(_trans_e_body)
    return run(h, r, t, entity_emb, rel2)


def kernel(h, r, t, entity_emb, relation_emb):
    h2 = h.astype(jnp.int32).reshape(NW, PW)
    r2 = r.astype(jnp.int32).reshape(NW, PW)
    t2 = t.astype(jnp.int32).reshape(NW, PW)
    return _trans_e(h2, r2, t2, entity_emb,
                    relation_emb.reshape(NR // 2, 2 * D))


# speedup vs baseline: 1.6148x; 1.0767x over previous
import functools

import jax
import jax.numpy as jnp
from jax import lax
from jax.experimental import pallas as pl
from jax.experimental.pallas import tpu as pltpu
from jax.experimental.pallas import tpu_sc as plsc

B = 16384
D = 64
NR = 1000
L = 16

_info = plsc.get_sparse_core_info()
NC = _info.num_cores
NS = _info.num_subcores
NW = NC * NS
PW = B // NW
CPR = 128
NCK = PW // CPR
CGRP = CPR // L


def _trans_e_body(h_hbm, r_hbm, t_hbm, ent_hbm, rel_hbm, out_hbm,
                  him, rim, tim, hb, tb, rloc, outv, sem):
    wid = lax.axis_index("s") * NC + lax.axis_index("c")

    pltpu.sync_copy(h_hbm.at[wid], him)
    pltpu.sync_copy(r_hbm.at[wid], rim)
    pltpu.sync_copy(t_hbm.at[wid], tim)
    def rel_body(gg, _):
        pltpu.async_copy(rel_hbm.at[pl.ds(gg * 8, 8)],
                         rloc.at[pl.ds(gg * 8, 8)], sem)
        return 0

    lax.fori_loop(0, (NR // 2) // 8, rel_body, 0)
    pltpu.async_copy(rel_hbm.at[pl.ds(496, 4)], rloc.at[pl.ds(496, 4)], sem)

    def rel_drain(gg, _):
        pltpu.make_async_copy(rel_hbm.at[pl.ds(0, 8)],
                              rloc.at[pl.ds(0, 8)], sem).wait()
        return 0

    lax.fori_loop(0, (NR // 2) // 8, rel_drain, 0)
    pltpu.make_async_copy(rel_hbm.at[pl.ds(496, 4)],
                          rloc.at[pl.ds(496, 4)], sem).wait()

    lane = lax.iota(jnp.int32, L)

    def chunk_body(k, _):
        base = k * CPR

        def fire_body(q, _):
            hv16 = him[pl.ds(base + q * L, L)]
            tv16 = tim[pl.ds(base + q * L, L)]
            for jj in range(L):
                i = q * L + jj
                pltpu.async_copy(ent_hbm.at[hv16[jj]], hb.at[i], sem)
                pltpu.async_copy(ent_hbm.at[tv16[jj]], tb.at[i], sem)
            return 0

        lax.fori_loop(0, CPR // L, fire_body, 0)

        def drain_body(i, _):
            pltpu.make_async_copy(ent_hbm.at[0], hb.at[0], sem).wait()
            pltpu.make_async_copy(ent_hbm.at[0], tb.at[0], sem).wait()
            return 0

        lax.fori_loop(0, CPR, drain_body, 0)

        def group_body(g, _):
            row = lane + g * L
            rv16 = rim[pl.ds(base + g * L, L)]
            rp16 = jax.lax.shift_right_logical(rv16, 1)
            ro16 = (rv16 & 1) * D
            acc = jnp.zeros((L,), jnp.float32)
            for c in range(D):
                col = jnp.full((L,), c, jnp.int32)
                hv = plsc.load_gather(hb, [row, col])
                tv = plsc.load_gather(tb, [row, col])
                rv = plsc.load_gather(rloc, [rp16, col + ro16])
                acc = acc + jnp.abs(hv + rv - tv)
            outv[pl.ds(base + g * L, L)] = acc
            return 0

        lax.fori_loop(0, CGRP, group_body, 0)
        return 0

    lax.fori_loop(0, NCK, chunk_body, 0)

    pltpu.sync_copy(outv, out_hbm.at[pl.ds(wid * PW, PW)])


@jax.jit
def _trans_e(h, r, t, entity_emb, rel2):
    mesh = plsc.VectorSubcoreMesh(core_axis_name="c", subcore_axis_name="s")
    run = functools.partial(
        pl.kernel,
        mesh=mesh,
        compiler_params=pltpu.CompilerParams(needs_layout_passes=False),
        out_type=jax.ShapeDtypeStruct((B,), jnp.float32),
        scratch_types=[
            pltpu.VMEM((PW,), jnp.int32),
            pltpu.VMEM((PW,), jnp.int32),
            pltpu.VMEM((PW,), jnp.int32),
            pltpu.VMEM((CPR, D), jnp.float32),
            pltpu.VMEM((CPR, D), jnp.float32),
            pltpu.VMEM((NR // 2, 2 * D), jnp.float32),
            pltpu.VMEM((PW,), jnp.float32),
            pltpu.SemaphoreType.DMA,
        ],
    )(_trans_e_body)
    return run(h, r, t, entity_emb, rel2)


def kernel(h, r, t, entity_emb, relation_emb):
    h2 = h.astype(jnp.int32).reshape(NW, PW)
    r2 = r.astype(jnp.int32).reshape(NW, PW)
    t2 = t.astype(jnp.int32).reshape(NW, PW)
    return _trans_e(h2, r2, t2, entity_emb,
                    relation_emb.reshape(NR // 2, 2 * D))
